# baseline (speedup 1.0000x reference)
import functools

import jax
import jax.numpy as jnp
from jax import lax
from jax.experimental import pallas as pl
from jax.experimental.pallas import tpu as pltpu
from jax.experimental.pallas import tpu_sc as plsc

NP_ = 576
PD_ = 768
B_ = 256

NC_ = 2
NS_ = 16
NW_ = NC_ * NS_
PP_ = NP_ // NW_
CHUNK_ = PP_ * PD_

_mesh = plsc.VectorSubcoreMesh(core_axis_name="c", subcore_axis_name="s")


@functools.partial(
    pl.kernel,
    out_type=jax.ShapeDtypeStruct((B_ * NP_ * PD_,), jnp.float32),
    mesh=_mesh,
    scratch_types=[
        pltpu.VMEM((CHUNK_,), jnp.float32),
        pltpu.VMEM((CHUNK_,), jnp.float32),
    ],
)
def _sc_add(x_hbm, t_hbm, out_hbm, t_v, x_v):
    w = lax.axis_index("s") * NC_ + lax.axis_index("c")
    t0 = w * CHUNK_
    pltpu.sync_copy(t_hbm.at[pl.ds(t0, CHUNK_)], t_v)

    def per_batch(b, carry):
        base = b * (NP_ * PD_) + t0
        pltpu.sync_copy(x_hbm.at[pl.ds(base, CHUNK_)], x_v)

        def vbody(j, c2):
            col = j * 16
            t16 = t_v[pl.ds(col, 16)]
            plsc.addupdate(x_v.at[pl.ds(col, 16)], t16)
            return c2

        lax.fori_loop(0, CHUNK_ // 16, vbody, 0)
        pltpu.sync_copy(x_v, out_hbm.at[pl.ds(base, CHUNK_)])
        return carry

    lax.fori_loop(0, B_, per_batch, 0)


def kernel(encoded_patches, pos_table):
    out = _sc_add(encoded_patches.reshape(-1), pos_table.reshape(-1))
    return out.reshape(B_, NP_, PD_)


# speedup vs baseline: 1.8624x; 1.8624x over previous
import functools

import jax
import jax.numpy as jnp
from jax import lax
from jax.experimental import pallas as pl
from jax.experimental.pallas import tpu as pltpu
from jax.experimental.pallas import tpu_sc as plsc

NP_ = 576
PD_ = 768
B_ = 256

NC_ = 2
NS_ = 16
NW_ = NC_ * NS_
PP_ = NP_ // NW_
CHUNK_ = PP_ * PD_
ROW_ = NP_ * PD_
NBUF_ = 8
NVREG_ = CHUNK_ // 16

_mesh = plsc.VectorSubcoreMesh(core_axis_name="c", subcore_axis_name="s")


@functools.partial(
    pl.kernel,
    out_type=jax.ShapeDtypeStruct((B_ * ROW_,), jnp.float32),
    mesh=_mesh,
    scratch_types=(
        [pltpu.VMEM((CHUNK_,), jnp.float32)]
        + [pltpu.VMEM((CHUNK_,), jnp.float32) for _ in range(NBUF_)]
        + [pltpu.SemaphoreType.DMA for _ in range(2 * NBUF_)]
    ),
)
def _sc_add(x_hbm, t_hbm, out_hbm, t_v, *bufs_and_sems):
    bufs = bufs_and_sems[:NBUF_]
    in_sems = bufs_and_sems[NBUF_:2 * NBUF_]
    out_sems = bufs_and_sems[2 * NBUF_:]

    w = lax.axis_index("s") * NC_ + lax.axis_index("c")
    t0 = w * CHUNK_
    pltpu.sync_copy(t_hbm.at[pl.ds(t0, CHUNK_)], t_v)

    def in_slice(s):
        return x_hbm.at[pl.ds(s * ROW_ + t0, CHUNK_)]

    def out_slice(s):
        return out_hbm.at[pl.ds(s * ROW_ + t0, CHUNK_)]

    for ph in range(NBUF_ // 2):
        pltpu.async_copy(in_slice(ph), bufs[ph], in_sems[ph])

    def step(i, carry):
        for ph in range(NBUF_):
            s = i * NBUF_ + ph
            bph = (ph + NBUF_ // 2) % NBUF_
            sprev = s - NBUF_ // 2
            snext = s + NBUF_ // 2

            def drain_buddy():
                pltpu.make_async_copy(bufs[bph], out_slice(sprev), out_sems[bph]).wait()

            def prefetch_buddy():
                pltpu.async_copy(in_slice(snext), bufs[bph], in_sems[bph])

            if ph < NBUF_ // 2:
                pl.when(i >= 1)(drain_buddy)
                prefetch_buddy()
            else:
                drain_buddy()
                pl.when(i < (B_ // NBUF_) - 1)(prefetch_buddy)

            pltpu.make_async_copy(in_slice(s), bufs[ph], in_sems[ph]).wait()

            buf = bufs[ph]

            @plsc.parallel_loop(0, NVREG_, unroll=8)
            def add_body(j):
                col = j * 16
                plsc.addupdate(buf.at[pl.ds(col, 16)], t_v[pl.ds(col, 16)])

            pltpu.async_copy(buf, out_slice(s), out_sems[ph])
        return carry

    lax.fori_loop(0, B_ // NBUF_, step, 0)

    for ph in range(NBUF_ // 2, NBUF_):
        s = B_ - NBUF_ + ph
        pltpu.make_async_copy(bufs[ph], out_slice(s), out_sems[ph]).wait()


def kernel(encoded_patches, pos_table):
    out = _sc_add(encoded_patches.reshape(-1), pos_table.reshape(-1))
    return out.reshape(B_, NP_, PD_)


# speedup vs baseline: 8.0913x; 4.3445x over previous
import jax
import jax.numpy as jnp
from jax.experimental import pallas as pl

NP_ = 576
PD_ = 768
B_ = 256
BLOCK_B = 4


def _add_kernel(x_ref, t_ref, o_ref):
    o_ref[...] = x_ref[...] + t_ref[...]


def kernel(encoded_patches, pos_table):
    grid = (B_ // BLOCK_B,)
    return pl.pallas_call(
        _add_kernel,
        grid=grid,
        in_specs=[
            pl.BlockSpec((BLOCK_B, NP_, PD_), lambda i: (i, 0, 0)),
            pl.BlockSpec((NP_, PD_), lambda i: (0, 0)),
        ],
        out_specs=pl.BlockSpec((BLOCK_B, NP_, PD_), lambda i: (i, 0, 0)),
        out_shape=jax.ShapeDtypeStruct((B_, NP_, PD_), jnp.float32),
    )(encoded_patches, pos_table)
